# baseline (speedup 1.0000x reference)
import functools

import jax
import jax.numpy as jnp
from jax import lax
from jax.experimental import pallas as pl
from jax.experimental.pallas import tpu as pltpu
from jax.experimental.pallas import tpu_sc as plsc

VOCAB = 1000000
HV = VOCAB // 2
D = 64
BATCH = 4096
SEQ = 200
B = BATCH * SEQ
PAD_ID = 0

NC = 2
NS = 16
L = 16
NW = NC * NS
BPW = B // NW
IPG = 128
C = 128
G = C // IPG
SG = 4
SGN = IPG // SG
CB = C // 128
CHUNKS = BPW // C
PAIRS = CHUNKS // 2
IDROWS = BPW // IPG
RP = IPG + 1

_mesh = plsc.VectorSubcoreMesh(core_axis_name="c", subcore_axis_name="s")


@functools.partial(
    pl.kernel,
    out_type=jax.ShapeDtypeStruct((SEQ, D // 8, BATCH // 128, 8, 128),
                                  jnp.float32),
    mesh=_mesh,
    scratch_types=[
        pltpu.VMEM((IDROWS, IPG), jnp.int32),
        pltpu.VMEM((G, IPG), jnp.int32),
        pltpu.VMEM((G, IPG), jnp.int32),
        pltpu.VMEM((C, RP), jnp.float32),
        pltpu.VMEM((C, RP), jnp.float32),
        pltpu.VMEM((D // 8, CB, 8, 128), jnp.float32),
        pltpu.VMEM((D // 8, CB, 8, 128), jnp.float32),
        pltpu.SemaphoreType.DMA,
        pltpu.SemaphoreType.DMA,
        pltpu.SemaphoreType.DMA,
        pltpu.SemaphoreType.DMA,
    ],
    compiler_params=pltpu.CompilerParams(needs_layout_passes=False),
)
def _embed_lookup(ids_hbm, table_hbm, out_hbm, idx_v, pidx0, pidx1,
                  rows0, rows1, tiles0, tiles1, sg0, sg1, sw0, sw1):
    wid = lax.axis_index("s") * NC + lax.axis_index("c")
    base = wid * BPW

    pltpu.sync_copy(ids_hbm.at[pl.ds(wid * IDROWS, IDROWS)], idx_v)

    iota16 = lax.iota(jnp.int32, L)

    def fire_gathers(k, pidx_v, rows_v, sg):
        for j in range(G):
            for t in range(IPG // L):
                sl = pl.ds(t * L, L)
                idv = idx_v[k * G + j, sl]
                pidx_v[j, sl] = idv - jnp.where(idv >= HV, HV, 0)
        for j in range(G):
            for q in range(SG):
                pltpu.async_copy(
                    table_hbm.at[pidx_v.at[j, pl.ds(q * SGN, SGN)]],
                    rows_v.at[pl.ds(j * IPG + q * SGN, SGN), pl.ds(0, IPG)],
                    sg,
                )

    def drain_gathers(pidx_v, rows_v, sg):
        for j in range(G):
            for q in range(SG):
                pltpu.make_async_copy(
                    table_hbm.at[pidx_v.at[j, pl.ds(q * SGN, SGN)]],
                    rows_v.at[pl.ds(j * IPG + q * SGN, SGN), pl.ds(0, IPG)],
                    sg,
                ).wait()

    def out_slice(k):
        flat0 = base + k * C
        s = flat0 >> 12
        tb0 = (flat0 & (BATCH - 1)) >> 7
        return out_hbm.at[s, :, pl.ds(tb0, CB)]

    def fire_write(k, tiles_v, sw):
        pltpu.async_copy(tiles_v, out_slice(k), sw)

    def drain_write(k, tiles_v, sw):
        pltpu.make_async_copy(tiles_v, out_slice(k), sw).wait()

    def transpose(k, rows_v, tiles_v):
        def grp_body(gi, c):
            tbi = gi >> 3
            lg = gi & 7
            sl = pl.ds(lg * L, L)
            idv = idx_v[k * G + tbi, sl]
            hv = jnp.where(idv >= HV, 1, 0)
            tokv = tbi * 128 + lg * L + iota16
            for te in range(D // 8):
                for se in range(8):
                    g = plsc.load_gather(
                        rows_v, [tokv, hv + 2 * (te * 8 + se)])
                    tiles_v[te, tbi, se, sl] = g
            return c

        lax.fori_loop(0, CB * 8, grp_body, 0)

    def fixup(k, tiles_v):
        vs = [
            idx_v[k * G + j, pl.ds(t * L, L)]
            for j in range(G)
            for t in range(IPG // L)
        ]
        mn_vec = functools.reduce(jnp.minimum, vs)
        mn = functools.reduce(jnp.minimum, [mn_vec[i] for i in range(L)])

        @pl.when(mn == PAD_ID)
        def _fixup():
            def grp_body(gi, c):
                tbi = gi >> 3
                lg = gi & 7
                sl = pl.ds(lg * L, L)
                idv = idx_v[k * G + tbi, sl]
                mvec = jnp.where(idv == PAD_ID, 0.0, 1.0).astype(jnp.float32)
                for te in range(D // 8):
                    for se in range(8):
                        tiles_v[te, tbi, se, sl] = (
                            tiles_v[te, tbi, se, sl] * mvec)
                return c

            lax.fori_loop(0, CB * 8, grp_body, 0)

    fire_gathers(0, pidx0, rows0, sg0)
    fire_gathers(1, pidx1, rows1, sg1)

    def pair_body(i, carry):
        a = 2 * i
        for (ko, pidx_v, rows_v, tiles_v, sg, sw) in (
                (0, pidx0, rows0, tiles0, sg0, sw0),
                (1, pidx1, rows1, tiles1, sg1, sw1)):
            k = a + ko
            drain_gathers(pidx_v, rows_v, sg)

            @pl.when(k >= 2)
            def _wait_prev_write(k=k, tiles_v=tiles_v, sw=sw):
                drain_write(k - 2, tiles_v, sw)

            transpose(k, rows_v, tiles_v)

            @pl.when(k + 2 < CHUNKS)
            def _refill(k=k, pidx_v=pidx_v, rows_v=rows_v, sg=sg):
                fire_gathers(k + 2, pidx_v, rows_v, sg)

            fixup(k, tiles_v)
            fire_write(k, tiles_v, sw)

        return carry

    lax.fori_loop(0, PAIRS, pair_body, 0)

    drain_write(CHUNKS - 2, tiles0, sw0)
    drain_write(CHUNKS - 1, tiles1, sw1)


def kernel(ids, embedding_mat):
    ids_sb = ids.T.reshape(B // IPG, IPG)
    table_pairs = embedding_mat.T.reshape(2 * D, HV).T
    out5 = _embed_lookup(ids_sb, table_pairs)
    return out5.transpose(2, 4, 0, 1, 3).reshape(BATCH, SEQ, D)


# speedup vs baseline: 1.1319x; 1.1319x over previous
import functools

import jax
import jax.numpy as jnp
from jax import lax
from jax.experimental import pallas as pl
from jax.experimental.pallas import tpu as pltpu
from jax.experimental.pallas import tpu_sc as plsc

VOCAB = 1000000
D = 64
BATCH = 4096
SEQ = 200
B = BATCH * SEQ
PAD_ID = 0

NC = 2
NS = 16
L = 16
NW = NC * NS
BPW = B // NW
IPG = 128
C = 512
G = C // IPG
CHUNKS = BPW // C
PAIRS = CHUNKS // 2
IDROWS = BPW // IPG

_mesh = plsc.VectorSubcoreMesh(core_axis_name="c", subcore_axis_name="s")


@functools.partial(
    pl.kernel,
    out_type=jax.ShapeDtypeStruct((BATCH, SEQ, D), jnp.float32),
    mesh=_mesh,
    scratch_types=[
        pltpu.VMEM((IDROWS, IPG), jnp.int32),
        pltpu.VMEM((C, D), jnp.float32),
        pltpu.VMEM((C, D), jnp.float32),
        pltpu.SemaphoreType.DMA,
        pltpu.SemaphoreType.DMA,
        pltpu.SemaphoreType.DMA,
        pltpu.SemaphoreType.DMA,
    ],
    compiler_params=pltpu.CompilerParams(use_tc_tiling_on_sc=False),
)
def _embed_lookup(ids_hbm, table_hbm, out_hbm, idx_v, rows0, rows1,
                  sg0, sg1, sw0, sw1):
    wid = lax.axis_index("s") * NC + lax.axis_index("c")
    base = wid * BPW

    pltpu.sync_copy(ids_hbm.at[pl.ds(wid * IDROWS, IDROWS)], idx_v)

    def fire_gathers(k, rows_v, sg):
        for j in range(G):
            pltpu.async_copy(
                table_hbm.at[idx_v.at[k * G + j]],
                rows_v.at[pl.ds(j * IPG, IPG)],
                sg,
            )

    def drain_gathers(k, rows_v, sg):
        for j in range(G):
            pltpu.make_async_copy(
                table_hbm.at[idx_v.at[k * G + j]],
                rows_v.at[pl.ds(j * IPG, IPG)],
                sg,
            ).wait()

    def out_slice(k):
        flat0 = base + k * C
        s = flat0 >> 12
        b0 = flat0 & (BATCH - 1)
        return out_hbm.at[pl.ds(b0, C), s]

    def fire_write(k, rows_v, sw):
        pltpu.async_copy(rows_v, out_slice(k), sw)

    def drain_write(k, rows_v, sw):
        pltpu.make_async_copy(rows_v, out_slice(k), sw).wait()

    def fixup(k, rows_v):
        vs = [
            idx_v[k * G + j, pl.ds(t * L, L)]
            for j in range(G)
            for t in range(IPG // L)
        ]
        mn_vec = functools.reduce(jnp.minimum, vs)
        mn = functools.reduce(jnp.minimum, [mn_vec[i] for i in range(L)])

        @pl.when(mn == PAD_ID)
        def _fixup():
            def grp_body(g, c):
                jq = g // (IPG // L)
                tq = g % (IPG // L)
                idv = idx_v[k * G + jq, pl.ds(tq * L, L)]
                mvec = jnp.where(idv == PAD_ID, 0.0, 1.0).astype(jnp.float32)
                for rl in range(L):
                    f = mvec[rl]
                    row = g * L + rl
                    for cb in range(D // L):
                        sl = pl.ds(cb * L, L)
                        rows_v[row, sl] = rows_v[row, sl] * f
                return c

            lax.fori_loop(0, C // L, grp_body, 0)

    fire_gathers(0, rows0, sg0)
    fire_gathers(1, rows1, sg1)

    def pair_body(i, carry):
        a = 2 * i
        for (ko, rows_v, sg, sw) in ((0, rows0, sg0, sw0),
                                     (1, rows1, sg1, sw1)):
            k = a + ko
            drain_gathers(k, rows_v, sg)
            fixup(k, rows_v)
            fire_write(k, rows_v, sw)

            @pl.when(k + 2 < CHUNKS)
            def _refill(k=k, rows_v=rows_v, sg=sg, sw=sw):
                drain_write(k, rows_v, sw)
                fire_gathers(k + 2, rows_v, sg)

        return carry

    lax.fori_loop(0, PAIRS, pair_body, 0)

    drain_write(CHUNKS - 2, rows0, sw0)
    drain_write(CHUNKS - 1, rows1, sw1)


def kernel(ids, embedding_mat):
    ids_sb = ids.T.reshape(B // IPG, IPG)
    return _embed_lookup(ids_sb, embedding_mat)
